# baseline (speedup 1.0000x reference)
import jax
import jax.numpy as jnp
from jax import lax
from jax.experimental import pallas as pl
from jax.experimental.pallas import tpu as pltpu
from jax.experimental.pallas import tpu_sc as plsc

_COMMIT = 0.99

_B, _T, _D, _K = 16, 1024, 32, 8192
_KBLK = 2048
_NKB = _K // _KBLK


_TB = 512
_NTB = _B * _T // _TB


def _dist_body(x_ref, w_ref, a_ref, b_ref, colf_ref, md_ref, mdt_ref,
               idx_ref):
    x = x_ref[...]
    x2 = x + x
    a = a_ref[0]
    rm = None
    for t in range(_NKB):
        w = w_ref[t * _KBLK:(t + 1) * _KBLK, :]
        c2 = lax.dot_general(x2, w, (((1,), (1,)), ((), ())),
                             preferred_element_type=jnp.float32)
        d = (a + b_ref[0, :, t * _KBLK:(t + 1) * _KBLK]) - c2
        tmin = jnp.min(d, axis=1, keepdims=True)
        colf = colf_ref[0, :, t * _KBLK:(t + 1) * _KBLK]
        ka = jnp.min(jnp.where(d == tmin, colf, jnp.float32(1e9)),
                     axis=1, keepdims=True)
        if t == 0:
            rm, ri = tmin, ka
        else:
            better = tmin < rm
            ri = jnp.where(better, ka, ri)
            rm = jnp.where(better, tmin, rm)
    md_ref[0] = rm
    mdt_ref[0] = rm.reshape(1, _TB)
    idx_ref[0] = ri.astype(jnp.int32)


def _argmin_dist(flat, W, a3, b3, colf3):
    return pl.pallas_call(
        _dist_body,
        grid=(_NTB,),
        in_specs=[
            pl.BlockSpec((_TB, _D), lambda i: (i, 0)),
            pl.BlockSpec((_K, _D), lambda i: (0, 0)),
            pl.BlockSpec((1, _TB, 1), lambda i: (i, 0, 0)),
            pl.BlockSpec((1, 1, _K), lambda i: (0, 0, 0)),
            pl.BlockSpec((1, 1, _K), lambda i: (0, 0, 0)),
        ],
        out_specs=[
            pl.BlockSpec((1, _TB, 1), lambda i: (i, 0, 0)),
            pl.BlockSpec((1, 1, _TB), lambda i: (i, 0, 0)),
            pl.BlockSpec((1, _TB, 1), lambda i: (i, 0, 0)),
        ],
        out_shape=[
            jax.ShapeDtypeStruct((_NTB, _TB, 1), jnp.float32),
            jax.ShapeDtypeStruct((_NTB, 1, _TB), jnp.float32),
            jax.ShapeDtypeStruct((_NTB, _TB, 1), jnp.int32),
        ],
    )(flat, W, a3, b3, colf3)


def _perm_body(ma_ref, mb_ref, idxf_ref, q04_ref, x_ref,
               qst_ref, loss_ref, perp_ref, q0_ref):
    i = pl.program_id(0)
    idxf = idxf_ref[0]

    @pl.when(i == 0)
    def _():
        m = jnp.bitwise_and(idxf, 3)
        q04 = q04_ref[...]
        q0 = jnp.where(
            m == 0, q04[:, 0:_D],
            jnp.where(m == 1, q04[:, _D:2 * _D],
                      jnp.where(m == 2, q04[:, 2 * _D:3 * _D],
                                q04[:, 3 * _D:])))
        hi = q0.astype(jnp.bfloat16)
        r1 = q0 - hi.astype(jnp.float32)
        mid = r1.astype(jnp.bfloat16)
        lo = (r1 - mid.astype(jnp.float32)).astype(jnp.bfloat16)
        q0_ref[...] = jnp.concatenate([hi, mid, lo], axis=1)
        ent = jnp.zeros((), jnp.float32)
        for t in range(_NKB):
            kk = lax.broadcasted_iota(jnp.int32, (_T, _KBLK), 1) + t * _KBLK
            eq = (idxf == kk).astype(jnp.float32)
            counts = jnp.sum(eq, axis=0)
            avg = counts * jnp.float32(1.0 / _T)
            ent = ent + jnp.sum(avg * jnp.log(avg + 1e-10))
        perp_ref[...] = jnp.exp(-ent).reshape(1, 1)

    ma = ma_ref[0]
    mb = mb_ref[0]
    ii = lax.broadcasted_iota(jnp.int32, (_T, _T), 0)
    jj = lax.broadcasted_iota(jnp.int32, (_T, _T), 1)
    before2 = (ma < mb) | ((ma == mb) & (ii < jj))
    rank_row = jnp.sum(before2.astype(jnp.int32), axis=0, keepdims=True)
    onehot = (ii == rank_row).astype(jnp.bfloat16)
    qp96 = lax.dot_general(onehot, q0_ref[...], (((1,), (0,)), ((), ())),
                           preferred_element_type=jnp.float32)
    qp = (qp96[:, 0:_D] + qp96[:, _D:2 * _D]) + qp96[:, 2 * _D:]
    x = x_ref[0]
    diff = qp - x
    qst_ref[0] = x + diff
    part = jnp.sum(diff * diff).reshape(1, 1)

    @pl.when(i == 0)
    def _():
        loss_ref[...] = part

    @pl.when(i != 0)
    def _():
        loss_ref[...] = loss_ref[...] + part

    @pl.when(i == _B - 1)
    def _():
        tot = loss_ref[...] / jnp.float32(_BT * _D)
        loss_ref[...] = tot + jnp.float32(_COMMIT) * tot


def _permute_apply(mdA, mdB, idxcol, q04, inputs):
    return pl.pallas_call(
        _perm_body,
        grid=(_B,),
        in_specs=[
            pl.BlockSpec((1, _T, 1), lambda i: (i, 0, 0)),
            pl.BlockSpec((1, 1, _T), lambda i: (i, 0, 0)),
            pl.BlockSpec((1, _T, 1), lambda i: (0, 0, 0)),
            pl.BlockSpec((_T, _DW), lambda i: (0, 0)),
            pl.BlockSpec((1, _T, _D), lambda i: (i, 0, 0)),
        ],
        out_specs=[
            pl.BlockSpec((1, _T, _D), lambda i: (i, 0, 0)),
            pl.BlockSpec((1, 1), lambda i: (0, 0)),
            pl.BlockSpec((1, 1), lambda i: (0, 0)),
        ],
        out_shape=[
            jax.ShapeDtypeStruct((_B, _T, _D), jnp.float32),
            jax.ShapeDtypeStruct((1, 1), jnp.float32),
            jax.ShapeDtypeStruct((1, 1), jnp.float32),
        ],
        scratch_shapes=[pltpu.VMEM((_T, 3 * _D), jnp.bfloat16)],
    )(mdA, mdB, idxcol, q04, inputs)


_NC, _NS, _L = 2, 16, 16
_NW = _NC * _NS
_BT = _B * _T
_GPW = _T // _NW
_DW = 4 * _D


def _sc_gather_body(table_hbm, idx_hbm, out_hbm, idx_v, idx4_v, rows_v, sem):
    wid = lax.axis_index("s") * _NC + lax.axis_index("c")
    pltpu.sync_copy(idx_hbm.at[pl.ds(wid * _GPW, _GPW)], idx_v)
    for j in range(_GPW // _L):
        idx4_v[0, pl.ds(j * _L, _L)] = jax.lax.shift_right_logical(
            idx_v[pl.ds(j * _L, _L)], 2)
    pltpu.async_copy(table_hbm.at[idx4_v.at[0]], rows_v, sem).wait()
    pltpu.sync_copy(rows_v, out_hbm.at[pl.ds(wid * _GPW, _GPW)])


def _gather_rows(W4, idx_flat):
    kern = pl.kernel(
        _sc_gather_body,
        mesh=plsc.VectorSubcoreMesh(core_axis_name="c", subcore_axis_name="s"),
        out_type=jax.ShapeDtypeStruct((_T, _DW), jnp.float32),
        scratch_types=[
            pltpu.VMEM((_GPW,), jnp.int32),
            pltpu.VMEM((1, _GPW), jnp.int32),
            pltpu.VMEM((_GPW, _DW), jnp.float32),
            pltpu.SemaphoreType.DMA,
        ],
    )
    return kern(W4, idx_flat)


def kernel(inputs, W):
    assert inputs.shape == (_B, _T, _D) and W.shape == (_K, _D)
    flat = inputs.reshape(-1, _D)
    a3 = jnp.sum(flat ** 2, axis=1, keepdims=True).reshape(_NTB, _TB, 1)
    b3 = jnp.sum(W ** 2, axis=1).reshape(1, 1, _K)
    colf3 = jnp.arange(_K, dtype=jnp.float32).reshape(1, 1, _K)
    md3, mdt3, idx3 = _argmin_dist(flat, W, a3, b3, colf3)
    mdA = md3.reshape(_B, _T, 1)
    mdB = mdt3.reshape(_B, 1, _T)
    idx_flat = idx3.reshape(-1)
    q04 = _gather_rows(W.reshape(_K // 4, _DW), idx_flat)
    qst, loss_arr, perp_arr = _permute_apply(
        mdA, mdB, idx3.reshape(_B, _T, 1), q04, inputs)
    return qst, loss_arr[0, 0], perp_arr[0, 0]


# speedup vs baseline: 1.1285x; 1.1285x over previous
import jax
import jax.numpy as jnp
from jax import lax
from jax.experimental import pallas as pl
from jax.experimental.pallas import tpu as pltpu
from jax.experimental.pallas import tpu_sc as plsc

_COMMIT = 0.99

_B, _T, _D, _K = 16, 1024, 32, 8192
_KBLK = 2048
_NKB = _K // _KBLK


_TB = 512
_NTB = _B * _T // _TB


_RC = 64
_NG = _KBLK // 128


def _dist_body(x_ref, w_ref, a_ref, b_ref, md_ref, mdt_ref, idx_ref):
    x = x_ref[...]
    x2 = x + x
    a = a_ref[0]
    lane_f = lax.broadcasted_iota(
        jnp.int32, (_RC, 128), 1).astype(jnp.float32)
    nch = _TB // _RC
    rm = [None] * nch
    ri = [None] * nch
    for t in range(_NKB):
        w = w_ref[t * _KBLK:(t + 1) * _KBLK, :]
        c2 = lax.dot_general(x2, w, (((1,), (1,)), ((), ())),
                             preferred_element_type=jnp.float32)
        bt = b_ref[0, :, t * _KBLK:(t + 1) * _KBLK]
        for rc in range(nch):
            rsl = slice(rc * _RC, (rc + 1) * _RC)
            ach = a[rsl, :]
            rmin = (ach + bt[:, 0:128]) - c2[rsl, 0:128]
            rbf = jnp.zeros((_RC, 128), jnp.float32)
            for g in range(1, _NG):
                dg = (ach + bt[:, g * 128:(g + 1) * 128]) \
                    - c2[rsl, g * 128:(g + 1) * 128]
                lt = dg < rmin
                rbf = jnp.where(lt, jnp.float32(g), rbf)
                rmin = jnp.minimum(rmin, dg)
            tminc = jnp.min(rmin, axis=1, keepdims=True)
            kc = rbf * jnp.float32(128) + lane_f + jnp.float32(t * _KBLK)
            kac = jnp.min(jnp.where(rmin == tminc, kc, jnp.float32(1e9)),
                          axis=1, keepdims=True)
            if t == 0:
                rm[rc], ri[rc] = tminc, kac
            else:
                better = tminc < rm[rc]
                ri[rc] = jnp.where(better, kac, ri[rc])
                rm[rc] = jnp.where(better, tminc, rm[rc])
    rmv = jnp.concatenate(rm, axis=0)
    riv = jnp.concatenate(ri, axis=0)
    md_ref[0] = rmv
    mdt_ref[0] = rmv.reshape(1, _TB)
    idx_ref[0] = riv.astype(jnp.int32)


def _argmin_dist(flat, W, a3, b3):
    return pl.pallas_call(
        _dist_body,
        grid=(_NTB,),
        in_specs=[
            pl.BlockSpec((_TB, _D), lambda i: (i, 0)),
            pl.BlockSpec((_K, _D), lambda i: (0, 0)),
            pl.BlockSpec((1, _TB, 1), lambda i: (i, 0, 0)),
            pl.BlockSpec((1, 1, _K), lambda i: (0, 0, 0)),
        ],
        out_specs=[
            pl.BlockSpec((1, _TB, 1), lambda i: (i, 0, 0)),
            pl.BlockSpec((1, 1, _TB), lambda i: (i, 0, 0)),
            pl.BlockSpec((1, _TB, 1), lambda i: (i, 0, 0)),
        ],
        out_shape=[
            jax.ShapeDtypeStruct((_NTB, _TB, 1), jnp.float32),
            jax.ShapeDtypeStruct((_NTB, 1, _TB), jnp.float32),
            jax.ShapeDtypeStruct((_NTB, _TB, 1), jnp.int32),
        ],
    )(flat, W, a3, b3)


def _perm_body(ma_ref, mb_ref, idxf_ref, q04_ref, x_ref,
               qst_ref, loss_ref, perp_ref, q0_ref):
    i = pl.program_id(0)
    idxf = idxf_ref[0]

    @pl.when(i == 0)
    def _():
        m = jnp.bitwise_and(idxf, 3)
        q04 = q04_ref[...]
        q0 = jnp.where(
            m == 0, q04[:, 0:_D],
            jnp.where(m == 1, q04[:, _D:2 * _D],
                      jnp.where(m == 2, q04[:, 2 * _D:3 * _D],
                                q04[:, 3 * _D:])))
        hi = q0.astype(jnp.bfloat16)
        r1 = q0 - hi.astype(jnp.float32)
        mid = r1.astype(jnp.bfloat16)
        lo = (r1 - mid.astype(jnp.float32)).astype(jnp.bfloat16)
        q0_ref[...] = jnp.concatenate([hi, mid, lo], axis=1)
        ent = jnp.zeros((), jnp.float32)
        for t in range(_NKB):
            kk = lax.broadcasted_iota(jnp.int32, (_T, _KBLK), 1) + t * _KBLK
            eq = (idxf == kk).astype(jnp.float32)
            counts = jnp.sum(eq, axis=0)
            avg = counts * jnp.float32(1.0 / _T)
            ent = ent + jnp.sum(avg * jnp.log(avg + 1e-10))
        perp_ref[...] = jnp.exp(-ent).reshape(1, 1)

    ma = ma_ref[0]
    mb = mb_ref[0]
    ii = lax.broadcasted_iota(jnp.int32, (_T, _T), 0)
    jj = lax.broadcasted_iota(jnp.int32, (_T, _T), 1)
    before2 = (ma < mb) | ((ma == mb) & (ii < jj))
    rank_row = jnp.sum(before2.astype(jnp.int32), axis=0, keepdims=True)
    onehot = (ii == rank_row).astype(jnp.bfloat16)
    qp96 = lax.dot_general(onehot, q0_ref[...], (((1,), (0,)), ((), ())),
                           preferred_element_type=jnp.float32)
    qp = (qp96[:, 0:_D] + qp96[:, _D:2 * _D]) + qp96[:, 2 * _D:]
    x = x_ref[0]
    diff = qp - x
    qst_ref[0] = x + diff
    part = jnp.sum(diff * diff).reshape(1, 1)

    @pl.when(i == 0)
    def _():
        loss_ref[...] = part

    @pl.when(i != 0)
    def _():
        loss_ref[...] = loss_ref[...] + part

    @pl.when(i == _B - 1)
    def _():
        tot = loss_ref[...] / jnp.float32(_BT * _D)
        loss_ref[...] = tot + jnp.float32(_COMMIT) * tot


def _permute_apply(mdA, mdB, idxcol, q04, inputs):
    return pl.pallas_call(
        _perm_body,
        grid=(_B,),
        in_specs=[
            pl.BlockSpec((1, _T, 1), lambda i: (i, 0, 0)),
            pl.BlockSpec((1, 1, _T), lambda i: (i, 0, 0)),
            pl.BlockSpec((1, _T, 1), lambda i: (0, 0, 0)),
            pl.BlockSpec((_T, _DW), lambda i: (0, 0)),
            pl.BlockSpec((1, _T, _D), lambda i: (i, 0, 0)),
        ],
        out_specs=[
            pl.BlockSpec((1, _T, _D), lambda i: (i, 0, 0)),
            pl.BlockSpec((1, 1), lambda i: (0, 0)),
            pl.BlockSpec((1, 1), lambda i: (0, 0)),
        ],
        out_shape=[
            jax.ShapeDtypeStruct((_B, _T, _D), jnp.float32),
            jax.ShapeDtypeStruct((1, 1), jnp.float32),
            jax.ShapeDtypeStruct((1, 1), jnp.float32),
        ],
        scratch_shapes=[pltpu.VMEM((_T, 3 * _D), jnp.bfloat16)],
    )(mdA, mdB, idxcol, q04, inputs)


_NC, _NS, _L = 2, 16, 16
_NW = _NC * _NS
_BT = _B * _T
_GPW = _T // _NW
_DW = 4 * _D


def _sc_gather_body(table_hbm, idx_hbm, out_hbm, idx_v, idx4_v, rows_v, sem):
    wid = lax.axis_index("s") * _NC + lax.axis_index("c")
    pltpu.sync_copy(idx_hbm.at[pl.ds(wid * _GPW, _GPW)], idx_v)
    for j in range(_GPW // _L):
        idx4_v[0, pl.ds(j * _L, _L)] = jax.lax.shift_right_logical(
            idx_v[pl.ds(j * _L, _L)], 2)
    pltpu.async_copy(table_hbm.at[idx4_v.at[0]], rows_v, sem).wait()
    pltpu.sync_copy(rows_v, out_hbm.at[pl.ds(wid * _GPW, _GPW)])


def _gather_rows(W4, idx_flat):
    kern = pl.kernel(
        _sc_gather_body,
        mesh=plsc.VectorSubcoreMesh(core_axis_name="c", subcore_axis_name="s"),
        out_type=jax.ShapeDtypeStruct((_T, _DW), jnp.float32),
        scratch_types=[
            pltpu.VMEM((_GPW,), jnp.int32),
            pltpu.VMEM((1, _GPW), jnp.int32),
            pltpu.VMEM((_GPW, _DW), jnp.float32),
            pltpu.SemaphoreType.DMA,
        ],
    )
    return kern(W4, idx_flat)


def kernel(inputs, W):
    assert inputs.shape == (_B, _T, _D) and W.shape == (_K, _D)
    flat = inputs.reshape(-1, _D)
    a3 = jnp.sum(flat ** 2, axis=1, keepdims=True).reshape(_NTB, _TB, 1)
    b3 = jnp.sum(W ** 2, axis=1).reshape(1, 1, _K)
    md3, mdt3, idx3 = _argmin_dist(flat, W, a3, b3)
    mdA = md3.reshape(_B, _T, 1)
    mdB = mdt3.reshape(_B, 1, _T)
    idx_flat = idx3.reshape(-1)
    q04 = _gather_rows(W.reshape(_K // 4, _DW), idx_flat)
    qst, loss_arr, perp_arr = _permute_apply(
        mdA, mdB, idx3.reshape(_B, _T, 1), q04, inputs)
    return qst, loss_arr[0, 0], perp_arr[0, 0]
